# Initial kernel scaffold; baseline (speedup 1.0000x reference)
import jax
import jax.numpy as jnp
from jax.experimental import pallas as pl


def kernel(x, t_table, c_table, W, b):
    raise NotImplementedError("write your pallas kernel here")


# speedup vs baseline: 9.5718x; 9.5718x over previous
import jax
import jax.numpy as jnp
from jax.experimental import pallas as pl
from jax.experimental.pallas import tpu as pltpu

CHANS = 64
SAMPLES = 1024
DIM = 128
ST = 128


def _body(x_ref, t_ref, c_ref, w_ref, b_ref, o_ref):
    xb = x_ref[0].T
    cb = c_ref[...] + b_ref[...]
    w = w_ref[...]
    t = t_ref[...]
    o_ref[0] = (xb[:, :, None] * w[None, :, :]
                + cb[None, :, :]
                + t[:, None, :])


def kernel(x, t_table, c_table, W, b):
    B = x.shape[0]
    xs = x[:, 0]
    grid = (B, SAMPLES // ST)
    out = pl.pallas_call(
        _body,
        grid=grid,
        in_specs=[
            pl.BlockSpec((1, CHANS, ST), lambda bi, si: (bi, 0, si)),
            pl.BlockSpec((ST, DIM), lambda bi, si: (si, 0)),
            pl.BlockSpec((CHANS, DIM), lambda bi, si: (0, 0)),
            pl.BlockSpec((1, DIM), lambda bi, si: (0, 0)),
            pl.BlockSpec((1, DIM), lambda bi, si: (0, 0)),
        ],
        out_specs=pl.BlockSpec((1, ST, CHANS, DIM), lambda bi, si: (bi, si, 0, 0)),
        out_shape=jax.ShapeDtypeStruct((B, SAMPLES, CHANS, DIM), jnp.float32),
    )(xs, t_table, c_table, W.T, b.reshape(1, DIM))
    return out.reshape(B, SAMPLES * CHANS, DIM)
